# Initial kernel scaffold; baseline (speedup 1.0000x reference)
import jax
import jax.numpy as jnp
from jax.experimental import pallas as pl


def kernel(x_user, x_movie, rating_edge_index, ratedby_edge_index, W0_rating, b0_rating, W0_ratedby, b0_ratedby, W1_rating, b1_rating, W1_ratedby, b1_ratedby):
    raise NotImplementedError("write your pallas kernel here")


# speedup vs baseline: 11.7537x; 11.7537x over previous
import functools

import jax
import jax.numpy as jnp
from jax import lax
from jax.experimental import pallas as pl
from jax.experimental.pallas import tpu as pltpu
from jax.experimental.pallas import tpu_sc as plsc

N = 50000
NPAD = 51200
RPT = NPAD // 16
D = 24
E = 1600000
CH = 128
GRP = 4
NTILE = 16
GROUPS = -(-E // (NTILE * GRP * CH))
ROWS_PER_TILE = GROUPS * GRP
NCR = ROWS_PER_TILE * NTILE
EP = NCR * CH
BLK = 1024
NBLK = NPAD // BLK

_mesh = plsc.VectorSubcoreMesh(core_axis_name="c", subcore_axis_name="s")
_f32 = jnp.float32
_sc_params = pltpu.CompilerParams(use_tc_tiling_on_sc=False)


@functools.partial(
    pl.kernel,
    out_type=[jax.ShapeDtypeStruct((NPAD,), _f32) for _ in range(4)],
    mesh=_mesh,
    scratch_types=[
        pltpu.VMEM((GRP, CH), jnp.int32),
        pltpu.VMEM((CH,), _f32),
        pltpu.VMEM((RPT,), _f32),
        pltpu.VMEM_SHARED((NPAD,), _f32),
        pltpu.VMEM_SHARED((NPAD,), _f32),
    ],
    compiler_params=_sc_params,
)
def _hist_kernel(rs_hbm, rd_hbm, rbs_hbm, rbd_hbm,
                 d_rs, d_rd, d_rbs, d_rbd,
                 idx_v, ones_v, zero_v, hist_a, hist_b):
    cid = lax.axis_index("c")
    sid = lax.axis_index("s")

    for j in range(CH // 16):
        ones_v[pl.ds(j * 16, 16)] = jnp.ones((16,), _f32)

    def _z(i, _):
        zero_v[pl.ds(i * 16, 16)] = jnp.zeros((16,), _f32)
        return _
    lax.fori_loop(0, RPT // 16, _z, None)

    sl = pl.ds(sid * RPT, RPT)
    pltpu.sync_copy(zero_v, hist_a.at[sl])
    pltpu.sync_copy(zero_v, hist_b.at[sl])
    plsc.subcore_barrier()

    def _hist_pair(src_hbm, dst_hbm):
        def _g(g, _):
            base = sid * ROWS_PER_TILE + g * GRP
            pltpu.sync_copy(src_hbm.at[pl.ds(base, GRP)], idx_v)
            for j in range(GRP):
                pltpu.sync_copy(ones_v, hist_a.at[idx_v.at[j]], add=True)
            pltpu.sync_copy(dst_hbm.at[pl.ds(base, GRP)], idx_v)
            for j in range(GRP):
                pltpu.sync_copy(ones_v, hist_b.at[idx_v.at[j]], add=True)
            return _
        lax.fori_loop(0, GROUPS, _g, None)
        plsc.subcore_barrier()

    pl.when(cid == 0)(lambda: _hist_pair(rs_hbm, rd_hbm))
    pl.when(cid == 1)(lambda: _hist_pair(rbs_hbm, rbd_hbm))

    @pl.when(cid == 0)
    def _():
        pltpu.sync_copy(hist_a.at[sl], d_rs.at[sl])
        pltpu.sync_copy(hist_b.at[sl], d_rd.at[sl])

    @pl.when(cid == 1)
    def _():
        pltpu.sync_copy(hist_a.at[sl], d_rbs.at[sl])
        pltpu.sync_copy(hist_b.at[sl], d_rbd.at[sl])


@functools.partial(
    pl.kernel,
    out_type=[jax.ShapeDtypeStruct((NPAD, D), _f32) for _ in range(2)],
    mesh=_mesh,
    scratch_types=[
        pltpu.VMEM((GRP, CH), jnp.int32),
        pltpu.VMEM((GRP, CH), jnp.int32),
        pltpu.VMEM((GRP, CH, D), _f32),
        pltpu.VMEM_SHARED((NPAD, D), _f32),
        pltpu.SemaphoreType.DMA,
    ],
    compiler_params=_sc_params,
)
def _conv_kernel(feat_u, feat_m, rs_hbm, rd_hbm, rbs_hbm, rbd_hbm, zeros_hbm,
                 out_m, out_u,
                 idx_s, idx_d, rows_v, agg, sem):
    cid = lax.axis_index("c")
    sid = lax.axis_index("s")

    def _conv(src_hbm, dst_hbm, feat_hbm, out_hbm):
        pltpu.sync_copy(zeros_hbm, agg.at[pl.ds(sid * RPT, RPT)])
        plsc.subcore_barrier()

        def _g(g, _):
            base = sid * ROWS_PER_TILE + g * GRP
            pltpu.sync_copy(src_hbm.at[pl.ds(base, GRP)], idx_s)
            pltpu.sync_copy(dst_hbm.at[pl.ds(base, GRP)], idx_d)
            descs = [
                pltpu.async_copy(feat_hbm.at[idx_s.at[j]], rows_v.at[j], sem)
                for j in range(GRP)
            ]
            for dsc in descs:
                dsc.wait()
            for j in range(GRP):
                pltpu.sync_copy(rows_v.at[j], agg.at[idx_d.at[j]], add=True)
            return _
        lax.fori_loop(0, GROUPS, _g, None)
        plsc.subcore_barrier()

        sl = pl.ds(sid * RPT, RPT)
        pltpu.sync_copy(agg.at[sl], out_hbm.at[sl])

    pl.when(cid == 0)(lambda: _conv(rs_hbm, rd_hbm, feat_u, out_m))
    pl.when(cid == 1)(lambda: _conv(rbs_hbm, rbd_hbm, feat_m, out_u))


def _rnorm(d):
    return lax.rsqrt(jnp.maximum(d, 1.0))


def _prep_body(xu_ref, xm_ref, du_ref, dm_ref, fu_ref, fm_ref):
    nu = _rnorm(du_ref[:])
    nm = _rnorm(dm_ref[:])
    fu_ref[:] = jnp.concatenate(
        [xu_ref[:] * nu, jnp.zeros((BLK, D - 20), _f32)], axis=1)
    fm_ref[:] = xm_ref[:] * nm


_prep_call = pl.pallas_call(
    _prep_body,
    grid=(NBLK,),
    in_specs=[
        pl.BlockSpec((BLK, 20), lambda i: (i, 0)),
        pl.BlockSpec((BLK, 24), lambda i: (i, 0)),
        pl.BlockSpec((BLK, 1), lambda i: (i, 0)),
        pl.BlockSpec((BLK, 1), lambda i: (i, 0)),
    ],
    out_specs=[
        pl.BlockSpec((BLK, D), lambda i: (i, 0)),
        pl.BlockSpec((BLK, D), lambda i: (i, 0)),
    ],
    out_shape=[jax.ShapeDtypeStruct((NPAD, D), _f32) for _ in range(2)],
)


def _mid_body(am_ref, au_ref, ddm_ref, ddu_ref, dsu_ref, dsm_ref,
              w0r_ref, b0r_ref, w0rb_ref, b0rb_ref, w1r_ref, w1rb_ref,
              f1u_ref, f1m_ref):
    hm = jnp.dot(am_ref[:], w0r_ref[:], preferred_element_type=_f32)
    hm = jnp.maximum(hm * _rnorm(ddm_ref[:]) + b0r_ref[0:1, :], 0.0)
    hu = jnp.dot(au_ref[:], w0rb_ref[:], preferred_element_type=_f32)
    hu = jnp.maximum(hu * _rnorm(ddu_ref[:]) + b0rb_ref[0:1, :], 0.0)
    f1u = jnp.dot(hu * _rnorm(dsu_ref[:]), w1r_ref[:],
                  preferred_element_type=_f32)
    f1m = jnp.dot(hm * _rnorm(dsm_ref[:]), w1rb_ref[:],
                  preferred_element_type=_f32)
    z = jnp.zeros((BLK, D - 20), _f32)
    f1u_ref[:] = jnp.concatenate([f1u, z], axis=1)
    f1m_ref[:] = jnp.concatenate([f1m, z], axis=1)


_mid_call = pl.pallas_call(
    _mid_body,
    grid=(NBLK,),
    in_specs=[
        pl.BlockSpec((BLK, D), lambda i: (i, 0)),
        pl.BlockSpec((BLK, D), lambda i: (i, 0)),
        pl.BlockSpec((BLK, 1), lambda i: (i, 0)),
        pl.BlockSpec((BLK, 1), lambda i: (i, 0)),
        pl.BlockSpec((BLK, 1), lambda i: (i, 0)),
        pl.BlockSpec((BLK, 1), lambda i: (i, 0)),
        pl.BlockSpec((D, 40), lambda i: (0, 0)),
        pl.BlockSpec((8, 40), lambda i: (0, 0)),
        pl.BlockSpec((D, 40), lambda i: (0, 0)),
        pl.BlockSpec((8, 40), lambda i: (0, 0)),
        pl.BlockSpec((40, 20), lambda i: (0, 0)),
        pl.BlockSpec((40, 20), lambda i: (0, 0)),
    ],
    out_specs=[
        pl.BlockSpec((BLK, D), lambda i: (i, 0)),
        pl.BlockSpec((BLK, D), lambda i: (i, 0)),
    ],
    out_shape=[jax.ShapeDtypeStruct((NPAD, D), _f32) for _ in range(2)],
)


def _final_body(au_ref, am_ref, du_ref, dm_ref, bu_ref, bm_ref,
                hu_ref, hm_ref):
    hu_ref[:] = au_ref[:][:, :20] * _rnorm(du_ref[:]) + bu_ref[0:1, :]
    hm_ref[:] = am_ref[:][:, :20] * _rnorm(dm_ref[:]) + bm_ref[0:1, :]


_final_call = pl.pallas_call(
    _final_body,
    grid=(NBLK,),
    in_specs=[
        pl.BlockSpec((BLK, D), lambda i: (i, 0)),
        pl.BlockSpec((BLK, D), lambda i: (i, 0)),
        pl.BlockSpec((BLK, 1), lambda i: (i, 0)),
        pl.BlockSpec((BLK, 1), lambda i: (i, 0)),
        pl.BlockSpec((8, 20), lambda i: (0, 0)),
        pl.BlockSpec((8, 20), lambda i: (0, 0)),
    ],
    out_specs=[
        pl.BlockSpec((BLK, 20), lambda i: (i, 0)),
        pl.BlockSpec((BLK, 20), lambda i: (i, 0)),
    ],
    out_shape=[jax.ShapeDtypeStruct((NPAD, 20), _f32) for _ in range(2)],
)


def _pad_edges(e):
    pad = jnp.full((EP - E,), N, jnp.int32)
    src = jnp.concatenate([e[0].astype(jnp.int32), pad]).reshape(NCR, CH)
    dst = jnp.concatenate([e[1].astype(jnp.int32), pad]).reshape(NCR, CH)
    return src, dst


def kernel(x_user, x_movie, rating_edge_index, ratedby_edge_index,
           W0_rating, b0_rating, W0_ratedby, b0_ratedby,
           W1_rating, b1_rating, W1_ratedby, b1_ratedby):
    rs, rd = _pad_edges(rating_edge_index)
    rbs, rbd = _pad_edges(ratedby_edge_index)

    xu = jnp.pad(x_user, ((0, NPAD - N), (0, 0)))
    xm = jnp.pad(x_movie, ((0, NPAD - N), (0, 0)))

    d_rs, d_rd, d_rbs, d_rbd = _hist_kernel(rs, rd, rbs, rbd)
    d_rs, d_rd = d_rs[:, None], d_rd[:, None]
    d_rbs, d_rbd = d_rbs[:, None], d_rbd[:, None]

    f0u, f0m = _prep_call(xu, xm, d_rs, d_rbs)
    zeros_hbm = jnp.zeros((RPT, D), _f32)
    a0m, a0u = _conv_kernel(f0u, f0m, rs, rd, rbs, rbd, zeros_hbm)

    w0r = jnp.pad(W0_rating, ((0, D - 20), (0, 0)))
    w0rb = jnp.pad(W0_ratedby, ((0, D - 24), (0, 0)))
    b0r = jnp.pad(b0_rating[None, :], ((0, 7), (0, 0)))
    b0rb = jnp.pad(b0_ratedby[None, :], ((0, 7), (0, 0)))
    b1r = jnp.pad(b1_rating[None, :], ((0, 7), (0, 0)))
    b1rb = jnp.pad(b1_ratedby[None, :], ((0, 7), (0, 0)))

    f1u, f1m = _mid_call(a0m, a0u, d_rd, d_rbd, d_rs, d_rbs,
                         w0r, b0r, w0rb, b0rb, W1_rating, W1_ratedby)
    a1m, a1u = _conv_kernel(f1u, f1m, rs, rd, rbs, rbd, zeros_hbm)

    hu2, hm2 = _final_call(a1u, a1m, d_rbd, d_rd, b1rb, b1r)
    return hu2[:N], hm2[:N]


# speedup vs baseline: 12.1339x; 1.0323x over previous
import functools

import jax
import jax.numpy as jnp
from jax import lax
from jax.experimental import pallas as pl
from jax.experimental.pallas import tpu as pltpu
from jax.experimental.pallas import tpu_sc as plsc

N = 50000
NPAD = 51200
RPT = NPAD // 16
D = 24
E = 1600000
CH = 128
GRP = 4
NTILE = 16
GROUPS = -(-E // (NTILE * GRP * CH))
ROWS_PER_TILE = GROUPS * GRP
NCR = ROWS_PER_TILE * NTILE
EP = NCR * CH
BLK = 1024
NBLK = NPAD // BLK

_mesh = plsc.VectorSubcoreMesh(core_axis_name="c", subcore_axis_name="s")
_f32 = jnp.float32
_sc_params = pltpu.CompilerParams(use_tc_tiling_on_sc=False)


@functools.partial(
    pl.kernel,
    out_type=[jax.ShapeDtypeStruct((NPAD,), _f32) for _ in range(4)],
    mesh=_mesh,
    scratch_types=[
        pltpu.VMEM((GRP, CH), jnp.int32),
        pltpu.VMEM((GRP, CH), jnp.int32),
        pltpu.VMEM((CH,), _f32),
        pltpu.VMEM((RPT,), _f32),
        pltpu.VMEM_SHARED((NPAD,), _f32),
        pltpu.VMEM_SHARED((NPAD,), _f32),
        pltpu.SemaphoreType.DMA,
        pltpu.SemaphoreType.DMA,
    ],
    compiler_params=_sc_params,
)
def _hist_kernel(rs_hbm, rd_hbm, rbs_hbm, rbd_hbm,
                 d_rs, d_rd, d_rbs, d_rbd,
                 idx_a, idx_b, ones_v, zero_v, hist_a, hist_b,
                 ssem_a, ssem_b):
    cid = lax.axis_index("c")
    sid = lax.axis_index("s")

    for j in range(CH // 16):
        ones_v[pl.ds(j * 16, 16)] = jnp.ones((16,), _f32)

    def _z(i, _):
        zero_v[pl.ds(i * 16, 16)] = jnp.zeros((16,), _f32)
        return _
    lax.fori_loop(0, RPT // 16, _z, None)

    sl = pl.ds(sid * RPT, RPT)
    pltpu.sync_copy(zero_v, hist_a.at[sl])
    pltpu.sync_copy(zero_v, hist_b.at[sl])
    plsc.subcore_barrier()

    def _hist_pair(src_hbm, dst_hbm):
        def _g(s, _):
            base = sid * ROWS_PER_TILE + s * GRP

            @pl.when(s > 0)
            def _():
                for j in range(GRP):
                    pltpu.make_async_copy(
                        ones_v, hist_a.at[idx_a.at[j]], ssem_a).wait()
            pltpu.sync_copy(src_hbm.at[pl.ds(base, GRP)], idx_a)
            for j in range(GRP):
                pltpu.async_copy(
                    ones_v, hist_a.at[idx_a.at[j]], ssem_a, add=True)

            @pl.when(s > 0)
            def _():
                for j in range(GRP):
                    pltpu.make_async_copy(
                        ones_v, hist_b.at[idx_b.at[j]], ssem_b).wait()
            pltpu.sync_copy(dst_hbm.at[pl.ds(base, GRP)], idx_b)
            for j in range(GRP):
                pltpu.async_copy(
                    ones_v, hist_b.at[idx_b.at[j]], ssem_b, add=True)
            return _
        lax.fori_loop(0, GROUPS, _g, None)
        for j in range(GRP):
            pltpu.make_async_copy(ones_v, hist_a.at[idx_a.at[j]], ssem_a).wait()
            pltpu.make_async_copy(ones_v, hist_b.at[idx_b.at[j]], ssem_b).wait()
        plsc.subcore_barrier()

    pl.when(cid == 0)(lambda: _hist_pair(rs_hbm, rd_hbm))
    pl.when(cid == 1)(lambda: _hist_pair(rbs_hbm, rbd_hbm))

    @pl.when(cid == 0)
    def _():
        pltpu.sync_copy(hist_a.at[sl], d_rs.at[sl])
        pltpu.sync_copy(hist_b.at[sl], d_rd.at[sl])

    @pl.when(cid == 1)
    def _():
        pltpu.sync_copy(hist_a.at[sl], d_rbs.at[sl])
        pltpu.sync_copy(hist_b.at[sl], d_rbd.at[sl])


@functools.partial(
    pl.kernel,
    out_type=[jax.ShapeDtypeStruct((NPAD, D), _f32) for _ in range(2)],
    mesh=_mesh,
    scratch_types=[
        pltpu.VMEM((GRP, CH), jnp.int32),
        pltpu.VMEM((GRP, CH), jnp.int32),
        pltpu.VMEM((2, GRP // 2, CH, D), _f32),
        pltpu.VMEM_SHARED((NPAD, D), _f32),
        pltpu.SemaphoreType.DMA,
        pltpu.SemaphoreType.DMA,
        pltpu.SemaphoreType.DMA,
    ],
    compiler_params=_sc_params,
)
def _conv_kernel(feat_u, feat_m, rs_hbm, rd_hbm, rbs_hbm, rbd_hbm, zeros_hbm,
                 out_m, out_u,
                 idx_s, idx_d, rows_v, agg, gsem, ssem0, ssem1):
    cid = lax.axis_index("c")
    sid = lax.axis_index("s")
    H = GRP // 2

    def _conv(src_hbm, dst_hbm, feat_hbm, out_hbm):
        pltpu.sync_copy(zeros_hbm, agg.at[pl.ds(sid * RPT, RPT)])
        plsc.subcore_barrier()

        def _g(s, _):
            base = sid * ROWS_PER_TILE + s * GRP

            @pl.when(s > 0)
            def _():
                for k in range(H):
                    pltpu.make_async_copy(
                        rows_v.at[1, k], agg.at[idx_d.at[H + k]], ssem1).wait()
            pltpu.sync_copy(src_hbm.at[pl.ds(base, GRP)], idx_s)
            pltpu.sync_copy(dst_hbm.at[pl.ds(base, GRP)], idx_d)

            ga = [
                pltpu.async_copy(feat_hbm.at[idx_s.at[k]], rows_v.at[0, k],
                                 gsem)
                for k in range(H)
            ]
            for dsc in ga:
                dsc.wait()
            for k in range(H):
                pltpu.async_copy(rows_v.at[0, k], agg.at[idx_d.at[k]],
                                 ssem0, add=True)

            gb = [
                pltpu.async_copy(feat_hbm.at[idx_s.at[H + k]],
                                 rows_v.at[1, k], gsem)
                for k in range(H)
            ]
            for dsc in gb:
                dsc.wait()
            for k in range(H):
                pltpu.make_async_copy(
                    rows_v.at[0, k], agg.at[idx_d.at[k]], ssem0).wait()
            for k in range(H):
                pltpu.async_copy(rows_v.at[1, k], agg.at[idx_d.at[H + k]],
                                 ssem1, add=True)
            return _
        lax.fori_loop(0, GROUPS, _g, None)
        for k in range(H):
            pltpu.make_async_copy(
                rows_v.at[1, k], agg.at[idx_d.at[H + k]], ssem1).wait()
        plsc.subcore_barrier()

        sl = pl.ds(sid * RPT, RPT)
        pltpu.sync_copy(agg.at[sl], out_hbm.at[sl])

    pl.when(cid == 0)(lambda: _conv(rs_hbm, rd_hbm, feat_u, out_m))
    pl.when(cid == 1)(lambda: _conv(rbs_hbm, rbd_hbm, feat_m, out_u))


def _rnorm(d):
    return lax.rsqrt(jnp.maximum(d, 1.0))


def _prep_body(xu_ref, xm_ref, du_ref, dm_ref, fu_ref, fm_ref):
    nu = _rnorm(du_ref[:])
    nm = _rnorm(dm_ref[:])
    fu_ref[:] = jnp.concatenate(
        [xu_ref[:] * nu, jnp.zeros((BLK, D - 20), _f32)], axis=1)
    fm_ref[:] = xm_ref[:] * nm


_prep_call = pl.pallas_call(
    _prep_body,
    grid=(NBLK,),
    in_specs=[
        pl.BlockSpec((BLK, 20), lambda i: (i, 0)),
        pl.BlockSpec((BLK, 24), lambda i: (i, 0)),
        pl.BlockSpec((BLK, 1), lambda i: (i, 0)),
        pl.BlockSpec((BLK, 1), lambda i: (i, 0)),
    ],
    out_specs=[
        pl.BlockSpec((BLK, D), lambda i: (i, 0)),
        pl.BlockSpec((BLK, D), lambda i: (i, 0)),
    ],
    out_shape=[jax.ShapeDtypeStruct((NPAD, D), _f32) for _ in range(2)],
)


def _mid_body(am_ref, au_ref, ddm_ref, ddu_ref, dsu_ref, dsm_ref,
              w0r_ref, b0r_ref, w0rb_ref, b0rb_ref, w1r_ref, w1rb_ref,
              f1u_ref, f1m_ref):
    hm = jnp.dot(am_ref[:], w0r_ref[:], preferred_element_type=_f32)
    hm = jnp.maximum(hm * _rnorm(ddm_ref[:]) + b0r_ref[0:1, :], 0.0)
    hu = jnp.dot(au_ref[:], w0rb_ref[:], preferred_element_type=_f32)
    hu = jnp.maximum(hu * _rnorm(ddu_ref[:]) + b0rb_ref[0:1, :], 0.0)
    f1u = jnp.dot(hu * _rnorm(dsu_ref[:]), w1r_ref[:],
                  preferred_element_type=_f32)
    f1m = jnp.dot(hm * _rnorm(dsm_ref[:]), w1rb_ref[:],
                  preferred_element_type=_f32)
    z = jnp.zeros((BLK, D - 20), _f32)
    f1u_ref[:] = jnp.concatenate([f1u, z], axis=1)
    f1m_ref[:] = jnp.concatenate([f1m, z], axis=1)


_mid_call = pl.pallas_call(
    _mid_body,
    grid=(NBLK,),
    in_specs=[
        pl.BlockSpec((BLK, D), lambda i: (i, 0)),
        pl.BlockSpec((BLK, D), lambda i: (i, 0)),
        pl.BlockSpec((BLK, 1), lambda i: (i, 0)),
        pl.BlockSpec((BLK, 1), lambda i: (i, 0)),
        pl.BlockSpec((BLK, 1), lambda i: (i, 0)),
        pl.BlockSpec((BLK, 1), lambda i: (i, 0)),
        pl.BlockSpec((D, 40), lambda i: (0, 0)),
        pl.BlockSpec((8, 40), lambda i: (0, 0)),
        pl.BlockSpec((D, 40), lambda i: (0, 0)),
        pl.BlockSpec((8, 40), lambda i: (0, 0)),
        pl.BlockSpec((40, 20), lambda i: (0, 0)),
        pl.BlockSpec((40, 20), lambda i: (0, 0)),
    ],
    out_specs=[
        pl.BlockSpec((BLK, D), lambda i: (i, 0)),
        pl.BlockSpec((BLK, D), lambda i: (i, 0)),
    ],
    out_shape=[jax.ShapeDtypeStruct((NPAD, D), _f32) for _ in range(2)],
)


def _final_body(au_ref, am_ref, du_ref, dm_ref, bu_ref, bm_ref,
                hu_ref, hm_ref):
    hu_ref[:] = au_ref[:][:, :20] * _rnorm(du_ref[:]) + bu_ref[0:1, :]
    hm_ref[:] = am_ref[:][:, :20] * _rnorm(dm_ref[:]) + bm_ref[0:1, :]


_final_call = pl.pallas_call(
    _final_body,
    grid=(NBLK,),
    in_specs=[
        pl.BlockSpec((BLK, D), lambda i: (i, 0)),
        pl.BlockSpec((BLK, D), lambda i: (i, 0)),
        pl.BlockSpec((BLK, 1), lambda i: (i, 0)),
        pl.BlockSpec((BLK, 1), lambda i: (i, 0)),
        pl.BlockSpec((8, 20), lambda i: (0, 0)),
        pl.BlockSpec((8, 20), lambda i: (0, 0)),
    ],
    out_specs=[
        pl.BlockSpec((BLK, 20), lambda i: (i, 0)),
        pl.BlockSpec((BLK, 20), lambda i: (i, 0)),
    ],
    out_shape=[jax.ShapeDtypeStruct((NPAD, 20), _f32) for _ in range(2)],
)


def _pad_edges(e):
    pad = jnp.full((EP - E,), N, jnp.int32)
    src = jnp.concatenate([e[0].astype(jnp.int32), pad]).reshape(NCR, CH)
    dst = jnp.concatenate([e[1].astype(jnp.int32), pad]).reshape(NCR, CH)
    return src, dst


def kernel(x_user, x_movie, rating_edge_index, ratedby_edge_index,
           W0_rating, b0_rating, W0_ratedby, b0_ratedby,
           W1_rating, b1_rating, W1_ratedby, b1_ratedby):
    rs, rd = _pad_edges(rating_edge_index)
    rbs, rbd = _pad_edges(ratedby_edge_index)

    xu = jnp.pad(x_user, ((0, NPAD - N), (0, 0)))
    xm = jnp.pad(x_movie, ((0, NPAD - N), (0, 0)))

    d_rs, d_rd, d_rbs, d_rbd = _hist_kernel(rs, rd, rbs, rbd)
    d_rs, d_rd = d_rs[:, None], d_rd[:, None]
    d_rbs, d_rbd = d_rbs[:, None], d_rbd[:, None]

    f0u, f0m = _prep_call(xu, xm, d_rs, d_rbs)
    zeros_hbm = jnp.zeros((RPT, D), _f32)
    a0m, a0u = _conv_kernel(f0u, f0m, rs, rd, rbs, rbd, zeros_hbm)

    w0r = jnp.pad(W0_rating, ((0, D - 20), (0, 0)))
    w0rb = jnp.pad(W0_ratedby, ((0, D - 24), (0, 0)))
    b0r = jnp.pad(b0_rating[None, :], ((0, 7), (0, 0)))
    b0rb = jnp.pad(b0_ratedby[None, :], ((0, 7), (0, 0)))
    b1r = jnp.pad(b1_rating[None, :], ((0, 7), (0, 0)))
    b1rb = jnp.pad(b1_ratedby[None, :], ((0, 7), (0, 0)))

    f1u, f1m = _mid_call(a0m, a0u, d_rd, d_rbd, d_rs, d_rbs,
                         w0r, b0r, w0rb, b0rb, W1_rating, W1_ratedby)
    a1m, a1u = _conv_kernel(f1u, f1m, rs, rd, rbs, rbd, zeros_hbm)

    hu2, hm2 = _final_call(a1u, a1m, d_rbd, d_rd, b1rb, b1r)
    return hu2[:N], hm2[:N]


# speedup vs baseline: 15.6303x; 1.2882x over previous
import functools

import jax
import jax.numpy as jnp
from jax import lax
from jax.experimental import pallas as pl
from jax.experimental.pallas import tpu as pltpu
from jax.experimental.pallas import tpu_sc as plsc

N = 50000
NPAD = 51200
RPT = NPAD // 16
D = 24
E = 1600000
CH = 128
GRP = 8
NTILE = 16
GROUPS = -(-E // (NTILE * GRP * CH))
ROWS_PER_TILE = GROUPS * GRP
NCR = ROWS_PER_TILE * NTILE
EP = NCR * CH
BLK = 1024
NBLK = NPAD // BLK

_mesh = plsc.VectorSubcoreMesh(core_axis_name="c", subcore_axis_name="s")
_f32 = jnp.float32
_sc_params = pltpu.CompilerParams(use_tc_tiling_on_sc=False)


@functools.partial(
    pl.kernel,
    out_type=[jax.ShapeDtypeStruct((NPAD,), _f32) for _ in range(4)],
    mesh=_mesh,
    scratch_types=[
        pltpu.VMEM((GRP, CH), jnp.int32),
        pltpu.VMEM((GRP, CH), jnp.int32),
        pltpu.VMEM((CH,), _f32),
        pltpu.VMEM((RPT,), _f32),
        pltpu.VMEM_SHARED((NPAD,), _f32),
        pltpu.VMEM_SHARED((NPAD,), _f32),
        pltpu.SemaphoreType.DMA,
        pltpu.SemaphoreType.DMA,
    ],
    compiler_params=_sc_params,
)
def _hist_kernel(rs_hbm, rd_hbm, rbs_hbm, rbd_hbm,
                 d_rs, d_rd, d_rbs, d_rbd,
                 idx_a, idx_b, ones_v, zero_v, hist_a, hist_b,
                 ssem_a, ssem_b):
    cid = lax.axis_index("c")
    sid = lax.axis_index("s")

    for j in range(CH // 16):
        ones_v[pl.ds(j * 16, 16)] = jnp.ones((16,), _f32)

    def _z(i, _):
        zero_v[pl.ds(i * 16, 16)] = jnp.zeros((16,), _f32)
        return _
    lax.fori_loop(0, RPT // 16, _z, None)

    sl = pl.ds(sid * RPT, RPT)
    pltpu.sync_copy(zero_v, hist_a.at[sl])
    pltpu.sync_copy(zero_v, hist_b.at[sl])
    plsc.subcore_barrier()

    def _hist_pair(src_hbm, dst_hbm):
        def _g(s, _):
            base = sid * ROWS_PER_TILE + s * GRP

            @pl.when(s > 0)
            def _():
                for j in range(GRP):
                    pltpu.make_async_copy(
                        ones_v, hist_a.at[idx_a.at[j]], ssem_a).wait()
            pltpu.sync_copy(src_hbm.at[pl.ds(base, GRP)], idx_a)
            for j in range(GRP):
                pltpu.async_copy(
                    ones_v, hist_a.at[idx_a.at[j]], ssem_a, add=True)

            @pl.when(s > 0)
            def _():
                for j in range(GRP):
                    pltpu.make_async_copy(
                        ones_v, hist_b.at[idx_b.at[j]], ssem_b).wait()
            pltpu.sync_copy(dst_hbm.at[pl.ds(base, GRP)], idx_b)
            for j in range(GRP):
                pltpu.async_copy(
                    ones_v, hist_b.at[idx_b.at[j]], ssem_b, add=True)
            return _
        lax.fori_loop(0, GROUPS, _g, None)
        for j in range(GRP):
            pltpu.make_async_copy(ones_v, hist_a.at[idx_a.at[j]], ssem_a).wait()
            pltpu.make_async_copy(ones_v, hist_b.at[idx_b.at[j]], ssem_b).wait()
        plsc.subcore_barrier()

    pl.when(cid == 0)(lambda: _hist_pair(rs_hbm, rd_hbm))
    pl.when(cid == 1)(lambda: _hist_pair(rbs_hbm, rbd_hbm))

    @pl.when(cid == 0)
    def _():
        pltpu.sync_copy(hist_a.at[sl], d_rs.at[sl])
        pltpu.sync_copy(hist_b.at[sl], d_rd.at[sl])

    @pl.when(cid == 1)
    def _():
        pltpu.sync_copy(hist_a.at[sl], d_rbs.at[sl])
        pltpu.sync_copy(hist_b.at[sl], d_rbd.at[sl])


@functools.partial(
    pl.kernel,
    out_type=[jax.ShapeDtypeStruct((NPAD, D), _f32) for _ in range(2)],
    mesh=_mesh,
    scratch_types=[
        pltpu.VMEM((GRP, CH), jnp.int32),
        pltpu.VMEM((GRP, CH), jnp.int32),
        pltpu.VMEM((2, GRP // 2, CH, D), _f32),
        pltpu.VMEM_SHARED((NPAD, D), _f32),
        pltpu.SemaphoreType.DMA,
        pltpu.SemaphoreType.DMA,
        pltpu.SemaphoreType.DMA,
    ],
    compiler_params=_sc_params,
)
def _conv_kernel(feat_u, feat_m, rs_hbm, rd_hbm, rbs_hbm, rbd_hbm, zeros_hbm,
                 out_m, out_u,
                 idx_s, idx_d, rows_v, agg, gsem, ssem0, ssem1):
    cid = lax.axis_index("c")
    sid = lax.axis_index("s")
    H = GRP // 2

    def _conv(src_hbm, dst_hbm, feat_hbm, out_hbm):
        pltpu.sync_copy(zeros_hbm, agg.at[pl.ds(sid * RPT, RPT)])
        plsc.subcore_barrier()

        def _g(s, _):
            base = sid * ROWS_PER_TILE + s * GRP

            @pl.when(s > 0)
            def _():
                for k in range(H):
                    pltpu.make_async_copy(
                        rows_v.at[1, k], agg.at[idx_d.at[H + k]], ssem1).wait()
            pltpu.sync_copy(src_hbm.at[pl.ds(base, GRP)], idx_s)
            pltpu.sync_copy(dst_hbm.at[pl.ds(base, GRP)], idx_d)

            ga = [
                pltpu.async_copy(feat_hbm.at[idx_s.at[k]], rows_v.at[0, k],
                                 gsem)
                for k in range(H)
            ]
            for dsc in ga:
                dsc.wait()
            for k in range(H):
                pltpu.async_copy(rows_v.at[0, k], agg.at[idx_d.at[k]],
                                 ssem0, add=True)

            gb = [
                pltpu.async_copy(feat_hbm.at[idx_s.at[H + k]],
                                 rows_v.at[1, k], gsem)
                for k in range(H)
            ]
            for dsc in gb:
                dsc.wait()
            for k in range(H):
                pltpu.make_async_copy(
                    rows_v.at[0, k], agg.at[idx_d.at[k]], ssem0).wait()
            for k in range(H):
                pltpu.async_copy(rows_v.at[1, k], agg.at[idx_d.at[H + k]],
                                 ssem1, add=True)
            return _
        lax.fori_loop(0, GROUPS, _g, None)
        for k in range(H):
            pltpu.make_async_copy(
                rows_v.at[1, k], agg.at[idx_d.at[H + k]], ssem1).wait()
        plsc.subcore_barrier()

        sl = pl.ds(sid * RPT, RPT)
        pltpu.sync_copy(agg.at[sl], out_hbm.at[sl])

    pl.when(cid == 0)(lambda: _conv(rs_hbm, rd_hbm, feat_u, out_m))
    pl.when(cid == 1)(lambda: _conv(rbs_hbm, rbd_hbm, feat_m, out_u))


def _rnorm(d):
    return lax.rsqrt(jnp.maximum(d, 1.0))


def _prep_body(xu_ref, xm_ref, du_ref, dm_ref, fu_ref, fm_ref):
    nu = _rnorm(du_ref[:])
    nm = _rnorm(dm_ref[:])
    fu_ref[:] = jnp.concatenate(
        [xu_ref[:] * nu, jnp.zeros((BLK, D - 20), _f32)], axis=1)
    fm_ref[:] = xm_ref[:] * nm


_prep_call = pl.pallas_call(
    _prep_body,
    grid=(NBLK,),
    in_specs=[
        pl.BlockSpec((BLK, 20), lambda i: (i, 0)),
        pl.BlockSpec((BLK, 24), lambda i: (i, 0)),
        pl.BlockSpec((BLK, 1), lambda i: (i, 0)),
        pl.BlockSpec((BLK, 1), lambda i: (i, 0)),
    ],
    out_specs=[
        pl.BlockSpec((BLK, D), lambda i: (i, 0)),
        pl.BlockSpec((BLK, D), lambda i: (i, 0)),
    ],
    out_shape=[jax.ShapeDtypeStruct((NPAD, D), _f32) for _ in range(2)],
)


def _mid_body(am_ref, au_ref, ddm_ref, ddu_ref, dsu_ref, dsm_ref,
              w0r_ref, b0r_ref, w0rb_ref, b0rb_ref, w1r_ref, w1rb_ref,
              f1u_ref, f1m_ref):
    hm = jnp.dot(am_ref[:], w0r_ref[:], preferred_element_type=_f32)
    hm = jnp.maximum(hm * _rnorm(ddm_ref[:]) + b0r_ref[0:1, :], 0.0)
    hu = jnp.dot(au_ref[:], w0rb_ref[:], preferred_element_type=_f32)
    hu = jnp.maximum(hu * _rnorm(ddu_ref[:]) + b0rb_ref[0:1, :], 0.0)
    f1u = jnp.dot(hu * _rnorm(dsu_ref[:]), w1r_ref[:],
                  preferred_element_type=_f32)
    f1m = jnp.dot(hm * _rnorm(dsm_ref[:]), w1rb_ref[:],
                  preferred_element_type=_f32)
    z = jnp.zeros((BLK, D - 20), _f32)
    f1u_ref[:] = jnp.concatenate([f1u, z], axis=1)
    f1m_ref[:] = jnp.concatenate([f1m, z], axis=1)


_mid_call = pl.pallas_call(
    _mid_body,
    grid=(NBLK,),
    in_specs=[
        pl.BlockSpec((BLK, D), lambda i: (i, 0)),
        pl.BlockSpec((BLK, D), lambda i: (i, 0)),
        pl.BlockSpec((BLK, 1), lambda i: (i, 0)),
        pl.BlockSpec((BLK, 1), lambda i: (i, 0)),
        pl.BlockSpec((BLK, 1), lambda i: (i, 0)),
        pl.BlockSpec((BLK, 1), lambda i: (i, 0)),
        pl.BlockSpec((D, 40), lambda i: (0, 0)),
        pl.BlockSpec((8, 40), lambda i: (0, 0)),
        pl.BlockSpec((D, 40), lambda i: (0, 0)),
        pl.BlockSpec((8, 40), lambda i: (0, 0)),
        pl.BlockSpec((40, 20), lambda i: (0, 0)),
        pl.BlockSpec((40, 20), lambda i: (0, 0)),
    ],
    out_specs=[
        pl.BlockSpec((BLK, D), lambda i: (i, 0)),
        pl.BlockSpec((BLK, D), lambda i: (i, 0)),
    ],
    out_shape=[jax.ShapeDtypeStruct((NPAD, D), _f32) for _ in range(2)],
)


def _final_body(au_ref, am_ref, du_ref, dm_ref, bu_ref, bm_ref,
                hu_ref, hm_ref):
    hu_ref[:] = au_ref[:][:, :20] * _rnorm(du_ref[:]) + bu_ref[0:1, :]
    hm_ref[:] = am_ref[:][:, :20] * _rnorm(dm_ref[:]) + bm_ref[0:1, :]


_final_call = pl.pallas_call(
    _final_body,
    grid=(NBLK,),
    in_specs=[
        pl.BlockSpec((BLK, D), lambda i: (i, 0)),
        pl.BlockSpec((BLK, D), lambda i: (i, 0)),
        pl.BlockSpec((BLK, 1), lambda i: (i, 0)),
        pl.BlockSpec((BLK, 1), lambda i: (i, 0)),
        pl.BlockSpec((8, 20), lambda i: (0, 0)),
        pl.BlockSpec((8, 20), lambda i: (0, 0)),
    ],
    out_specs=[
        pl.BlockSpec((BLK, 20), lambda i: (i, 0)),
        pl.BlockSpec((BLK, 20), lambda i: (i, 0)),
    ],
    out_shape=[jax.ShapeDtypeStruct((NPAD, 20), _f32) for _ in range(2)],
)


def _pad_edges(e):
    pad = jnp.full((EP - E,), N, jnp.int32)
    src = jnp.concatenate([e[0].astype(jnp.int32), pad]).reshape(NCR, CH)
    dst = jnp.concatenate([e[1].astype(jnp.int32), pad]).reshape(NCR, CH)
    return src, dst


def kernel(x_user, x_movie, rating_edge_index, ratedby_edge_index,
           W0_rating, b0_rating, W0_ratedby, b0_ratedby,
           W1_rating, b1_rating, W1_ratedby, b1_ratedby):
    rs, rd = _pad_edges(rating_edge_index)
    rbs, rbd = _pad_edges(ratedby_edge_index)

    xu = jnp.pad(x_user, ((0, NPAD - N), (0, 0)))
    xm = jnp.pad(x_movie, ((0, NPAD - N), (0, 0)))

    d_rs, d_rd, d_rbs, d_rbd = _hist_kernel(rs, rd, rbs, rbd)
    d_rs, d_rd = d_rs[:, None], d_rd[:, None]
    d_rbs, d_rbd = d_rbs[:, None], d_rbd[:, None]

    f0u, f0m = _prep_call(xu, xm, d_rs, d_rbs)
    zeros_hbm = jnp.zeros((RPT, D), _f32)
    a0m, a0u = _conv_kernel(f0u, f0m, rs, rd, rbs, rbd, zeros_hbm)

    w0r = jnp.pad(W0_rating, ((0, D - 20), (0, 0)))
    w0rb = jnp.pad(W0_ratedby, ((0, D - 24), (0, 0)))
    b0r = jnp.pad(b0_rating[None, :], ((0, 7), (0, 0)))
    b0rb = jnp.pad(b0_ratedby[None, :], ((0, 7), (0, 0)))
    b1r = jnp.pad(b1_rating[None, :], ((0, 7), (0, 0)))
    b1rb = jnp.pad(b1_ratedby[None, :], ((0, 7), (0, 0)))

    f1u, f1m = _mid_call(a0m, a0u, d_rd, d_rbd, d_rs, d_rbs,
                         w0r, b0r, w0rb, b0rb, W1_rating, W1_ratedby)
    a1m, a1u = _conv_kernel(f1u, f1m, rs, rd, rbs, rbd, zeros_hbm)

    hu2, hm2 = _final_call(a1u, a1m, d_rbd, d_rd, b1rb, b1r)
    return hu2[:N], hm2[:N]


# speedup vs baseline: 18.2283x; 1.1662x over previous
import functools

import jax
import jax.numpy as jnp
from jax import lax
from jax.experimental import pallas as pl
from jax.experimental.pallas import tpu as pltpu
from jax.experimental.pallas import tpu_sc as plsc

N = 50000
NPAD = 51200
RPT = NPAD // 16
D = 24
E = 1600000
CH = 128
GRP = 16
NTILE = 16
GROUPS = -(-E // (NTILE * GRP * CH))
ROWS_PER_TILE = GROUPS * GRP
NCR = ROWS_PER_TILE * NTILE
EP = NCR * CH
BLK = 1024
NBLK = NPAD // BLK

_mesh = plsc.VectorSubcoreMesh(core_axis_name="c", subcore_axis_name="s")
_f32 = jnp.float32
_sc_params = pltpu.CompilerParams(use_tc_tiling_on_sc=False)


@functools.partial(
    pl.kernel,
    out_type=[jax.ShapeDtypeStruct((NPAD,), _f32) for _ in range(4)],
    mesh=_mesh,
    scratch_types=[
        pltpu.VMEM((GRP, CH), jnp.int32),
        pltpu.VMEM((GRP, CH), jnp.int32),
        pltpu.VMEM((CH,), _f32),
        pltpu.VMEM((RPT,), _f32),
        pltpu.VMEM_SHARED((NPAD,), _f32),
        pltpu.VMEM_SHARED((NPAD,), _f32),
        pltpu.SemaphoreType.DMA,
        pltpu.SemaphoreType.DMA,
    ],
    compiler_params=_sc_params,
)
def _hist_kernel(rs_hbm, rd_hbm, rbs_hbm, rbd_hbm,
                 d_rs, d_rd, d_rbs, d_rbd,
                 idx_a, idx_b, ones_v, zero_v, hist_a, hist_b,
                 ssem_a, ssem_b):
    cid = lax.axis_index("c")
    sid = lax.axis_index("s")

    for j in range(CH // 16):
        ones_v[pl.ds(j * 16, 16)] = jnp.ones((16,), _f32)

    def _z(i, _):
        zero_v[pl.ds(i * 16, 16)] = jnp.zeros((16,), _f32)
        return _
    lax.fori_loop(0, RPT // 16, _z, None)

    sl = pl.ds(sid * RPT, RPT)
    pltpu.sync_copy(zero_v, hist_a.at[sl])
    pltpu.sync_copy(zero_v, hist_b.at[sl])
    plsc.subcore_barrier()

    def _hist_pair(src_hbm, dst_hbm):
        def _g(s, _):
            base = sid * ROWS_PER_TILE + s * GRP

            @pl.when(s > 0)
            def _():
                for j in range(GRP):
                    pltpu.make_async_copy(
                        ones_v, hist_a.at[idx_a.at[j]], ssem_a).wait()
            pltpu.sync_copy(src_hbm.at[pl.ds(base, GRP)], idx_a)
            for j in range(GRP):
                pltpu.async_copy(
                    ones_v, hist_a.at[idx_a.at[j]], ssem_a, add=True)

            @pl.when(s > 0)
            def _():
                for j in range(GRP):
                    pltpu.make_async_copy(
                        ones_v, hist_b.at[idx_b.at[j]], ssem_b).wait()
            pltpu.sync_copy(dst_hbm.at[pl.ds(base, GRP)], idx_b)
            for j in range(GRP):
                pltpu.async_copy(
                    ones_v, hist_b.at[idx_b.at[j]], ssem_b, add=True)
            return _
        lax.fori_loop(0, GROUPS, _g, None)
        for j in range(GRP):
            pltpu.make_async_copy(ones_v, hist_a.at[idx_a.at[j]], ssem_a).wait()
            pltpu.make_async_copy(ones_v, hist_b.at[idx_b.at[j]], ssem_b).wait()
        plsc.subcore_barrier()

    pl.when(cid == 0)(lambda: _hist_pair(rs_hbm, rd_hbm))
    pl.when(cid == 1)(lambda: _hist_pair(rbs_hbm, rbd_hbm))

    @pl.when(cid == 0)
    def _():
        pltpu.sync_copy(hist_a.at[sl], d_rs.at[sl])
        pltpu.sync_copy(hist_b.at[sl], d_rd.at[sl])

    @pl.when(cid == 1)
    def _():
        pltpu.sync_copy(hist_a.at[sl], d_rbs.at[sl])
        pltpu.sync_copy(hist_b.at[sl], d_rbd.at[sl])


@functools.partial(
    pl.kernel,
    out_type=[jax.ShapeDtypeStruct((NPAD, D), _f32) for _ in range(2)],
    mesh=_mesh,
    scratch_types=[
        pltpu.VMEM((GRP, CH), jnp.int32),
        pltpu.VMEM((GRP, CH), jnp.int32),
        pltpu.VMEM((2, GRP // 2, CH, D), _f32),
        pltpu.VMEM_SHARED((NPAD, D), _f32),
        pltpu.SemaphoreType.DMA,
        pltpu.SemaphoreType.DMA,
        pltpu.SemaphoreType.DMA,
    ],
    compiler_params=_sc_params,
)
def _conv_kernel(feat_u, feat_m, rs_hbm, rd_hbm, rbs_hbm, rbd_hbm, zeros_hbm,
                 out_m, out_u,
                 idx_s, idx_d, rows_v, agg, gsem, ssem0, ssem1):
    cid = lax.axis_index("c")
    sid = lax.axis_index("s")
    H = GRP // 2

    def _conv(src_hbm, dst_hbm, feat_hbm, out_hbm):
        pltpu.sync_copy(zeros_hbm, agg.at[pl.ds(sid * RPT, RPT)])
        plsc.subcore_barrier()

        def _g(s, _):
            base = sid * ROWS_PER_TILE + s * GRP

            @pl.when(s > 0)
            def _():
                for k in range(H):
                    pltpu.make_async_copy(
                        rows_v.at[1, k], agg.at[idx_d.at[H + k]], ssem1).wait()
            pltpu.sync_copy(src_hbm.at[pl.ds(base, GRP)], idx_s)
            pltpu.sync_copy(dst_hbm.at[pl.ds(base, GRP)], idx_d)

            ga = [
                pltpu.async_copy(feat_hbm.at[idx_s.at[k]], rows_v.at[0, k],
                                 gsem)
                for k in range(H)
            ]
            for dsc in ga:
                dsc.wait()
            for k in range(H):
                pltpu.async_copy(rows_v.at[0, k], agg.at[idx_d.at[k]],
                                 ssem0, add=True)

            gb = [
                pltpu.async_copy(feat_hbm.at[idx_s.at[H + k]],
                                 rows_v.at[1, k], gsem)
                for k in range(H)
            ]
            for dsc in gb:
                dsc.wait()
            for k in range(H):
                pltpu.make_async_copy(
                    rows_v.at[0, k], agg.at[idx_d.at[k]], ssem0).wait()
            for k in range(H):
                pltpu.async_copy(rows_v.at[1, k], agg.at[idx_d.at[H + k]],
                                 ssem1, add=True)
            return _
        lax.fori_loop(0, GROUPS, _g, None)
        for k in range(H):
            pltpu.make_async_copy(
                rows_v.at[1, k], agg.at[idx_d.at[H + k]], ssem1).wait()
        plsc.subcore_barrier()

        sl = pl.ds(sid * RPT, RPT)
        pltpu.sync_copy(agg.at[sl], out_hbm.at[sl])

    pl.when(cid == 0)(lambda: _conv(rs_hbm, rd_hbm, feat_u, out_m))
    pl.when(cid == 1)(lambda: _conv(rbs_hbm, rbd_hbm, feat_m, out_u))


def _rnorm(d):
    return lax.rsqrt(jnp.maximum(d, 1.0))


def _prep_body(xu_ref, xm_ref, du_ref, dm_ref, fu_ref, fm_ref):
    nu = _rnorm(du_ref[:])
    nm = _rnorm(dm_ref[:])
    fu_ref[:] = jnp.concatenate(
        [xu_ref[:] * nu, jnp.zeros((BLK, D - 20), _f32)], axis=1)
    fm_ref[:] = xm_ref[:] * nm


_prep_call = pl.pallas_call(
    _prep_body,
    grid=(NBLK,),
    in_specs=[
        pl.BlockSpec((BLK, 20), lambda i: (i, 0)),
        pl.BlockSpec((BLK, 24), lambda i: (i, 0)),
        pl.BlockSpec((BLK, 1), lambda i: (i, 0)),
        pl.BlockSpec((BLK, 1), lambda i: (i, 0)),
    ],
    out_specs=[
        pl.BlockSpec((BLK, D), lambda i: (i, 0)),
        pl.BlockSpec((BLK, D), lambda i: (i, 0)),
    ],
    out_shape=[jax.ShapeDtypeStruct((NPAD, D), _f32) for _ in range(2)],
)


def _mid_body(am_ref, au_ref, ddm_ref, ddu_ref, dsu_ref, dsm_ref,
              w0r_ref, b0r_ref, w0rb_ref, b0rb_ref, w1r_ref, w1rb_ref,
              f1u_ref, f1m_ref):
    hm = jnp.dot(am_ref[:], w0r_ref[:], preferred_element_type=_f32)
    hm = jnp.maximum(hm * _rnorm(ddm_ref[:]) + b0r_ref[0:1, :], 0.0)
    hu = jnp.dot(au_ref[:], w0rb_ref[:], preferred_element_type=_f32)
    hu = jnp.maximum(hu * _rnorm(ddu_ref[:]) + b0rb_ref[0:1, :], 0.0)
    f1u = jnp.dot(hu * _rnorm(dsu_ref[:]), w1r_ref[:],
                  preferred_element_type=_f32)
    f1m = jnp.dot(hm * _rnorm(dsm_ref[:]), w1rb_ref[:],
                  preferred_element_type=_f32)
    z = jnp.zeros((BLK, D - 20), _f32)
    f1u_ref[:] = jnp.concatenate([f1u, z], axis=1)
    f1m_ref[:] = jnp.concatenate([f1m, z], axis=1)


_mid_call = pl.pallas_call(
    _mid_body,
    grid=(NBLK,),
    in_specs=[
        pl.BlockSpec((BLK, D), lambda i: (i, 0)),
        pl.BlockSpec((BLK, D), lambda i: (i, 0)),
        pl.BlockSpec((BLK, 1), lambda i: (i, 0)),
        pl.BlockSpec((BLK, 1), lambda i: (i, 0)),
        pl.BlockSpec((BLK, 1), lambda i: (i, 0)),
        pl.BlockSpec((BLK, 1), lambda i: (i, 0)),
        pl.BlockSpec((D, 40), lambda i: (0, 0)),
        pl.BlockSpec((8, 40), lambda i: (0, 0)),
        pl.BlockSpec((D, 40), lambda i: (0, 0)),
        pl.BlockSpec((8, 40), lambda i: (0, 0)),
        pl.BlockSpec((40, 20), lambda i: (0, 0)),
        pl.BlockSpec((40, 20), lambda i: (0, 0)),
    ],
    out_specs=[
        pl.BlockSpec((BLK, D), lambda i: (i, 0)),
        pl.BlockSpec((BLK, D), lambda i: (i, 0)),
    ],
    out_shape=[jax.ShapeDtypeStruct((NPAD, D), _f32) for _ in range(2)],
)


def _final_body(au_ref, am_ref, du_ref, dm_ref, bu_ref, bm_ref,
                hu_ref, hm_ref):
    hu_ref[:] = au_ref[:][:, :20] * _rnorm(du_ref[:]) + bu_ref[0:1, :]
    hm_ref[:] = am_ref[:][:, :20] * _rnorm(dm_ref[:]) + bm_ref[0:1, :]


_final_call = pl.pallas_call(
    _final_body,
    grid=(NBLK,),
    in_specs=[
        pl.BlockSpec((BLK, D), lambda i: (i, 0)),
        pl.BlockSpec((BLK, D), lambda i: (i, 0)),
        pl.BlockSpec((BLK, 1), lambda i: (i, 0)),
        pl.BlockSpec((BLK, 1), lambda i: (i, 0)),
        pl.BlockSpec((8, 20), lambda i: (0, 0)),
        pl.BlockSpec((8, 20), lambda i: (0, 0)),
    ],
    out_specs=[
        pl.BlockSpec((BLK, 20), lambda i: (i, 0)),
        pl.BlockSpec((BLK, 20), lambda i: (i, 0)),
    ],
    out_shape=[jax.ShapeDtypeStruct((NPAD, 20), _f32) for _ in range(2)],
)


def _pad_edges(e):
    pad = jnp.full((EP - E,), N, jnp.int32)
    src = jnp.concatenate([e[0].astype(jnp.int32), pad]).reshape(NCR, CH)
    dst = jnp.concatenate([e[1].astype(jnp.int32), pad]).reshape(NCR, CH)
    return src, dst


def kernel(x_user, x_movie, rating_edge_index, ratedby_edge_index,
           W0_rating, b0_rating, W0_ratedby, b0_ratedby,
           W1_rating, b1_rating, W1_ratedby, b1_ratedby):
    rs, rd = _pad_edges(rating_edge_index)
    rbs, rbd = _pad_edges(ratedby_edge_index)

    xu = jnp.pad(x_user, ((0, NPAD - N), (0, 0)))
    xm = jnp.pad(x_movie, ((0, NPAD - N), (0, 0)))

    d_rs, d_rd, d_rbs, d_rbd = _hist_kernel(rs, rd, rbs, rbd)
    d_rs, d_rd = d_rs[:, None], d_rd[:, None]
    d_rbs, d_rbd = d_rbs[:, None], d_rbd[:, None]

    f0u, f0m = _prep_call(xu, xm, d_rs, d_rbs)
    zeros_hbm = jnp.zeros((RPT, D), _f32)
    a0m, a0u = _conv_kernel(f0u, f0m, rs, rd, rbs, rbd, zeros_hbm)

    w0r = jnp.pad(W0_rating, ((0, D - 20), (0, 0)))
    w0rb = jnp.pad(W0_ratedby, ((0, D - 24), (0, 0)))
    b0r = jnp.pad(b0_rating[None, :], ((0, 7), (0, 0)))
    b0rb = jnp.pad(b0_ratedby[None, :], ((0, 7), (0, 0)))
    b1r = jnp.pad(b1_rating[None, :], ((0, 7), (0, 0)))
    b1rb = jnp.pad(b1_ratedby[None, :], ((0, 7), (0, 0)))

    f1u, f1m = _mid_call(a0m, a0u, d_rd, d_rbd, d_rs, d_rbs,
                         w0r, b0r, w0rb, b0rb, W1_rating, W1_ratedby)
    a1m, a1u = _conv_kernel(f1u, f1m, rs, rd, rbs, rbd, zeros_hbm)

    hu2, hm2 = _final_call(a1u, a1m, d_rbd, d_rd, b1rb, b1r)
    return hu2[:N], hm2[:N]
